# baseline (speedup 1.0000x reference)
import numpy as np
import jax
import jax.numpy as jnp
from jax.experimental import pallas as pl

NQ = 16384
NP = 8192
QB = 512
CK = 256
NCK = NP // CK
KSEL = 32.0

_HI = jax.lax.Precision.HIGHEST


def _xyz_perm():
    q0, d0, s0, v0 = 0, 63, 72, 135
    perm = [q0, q0 + 1, q0 + 2, s0, s0 + 1, s0 + 2, v0, v0 + 1, v0 + 2, d0]
    sin_rows, cos_rows = [], []
    for f in range(10):
        sin_rows += [q0 + 3 + 6 * f + c for c in range(3)]
        sin_rows += [s0 + 3 + 6 * f + c for c in range(3)]
        sin_rows += [v0 + 3 + 6 * f + c for c in range(3)]
        cos_rows += [q0 + 6 + 6 * f + c for c in range(3)]
        cos_rows += [s0 + 6 + 6 * f + c for c in range(3)]
        cos_rows += [v0 + 6 + 6 * f + c for c in range(3)]
        if f < 4:
            sin_rows.append(d0 + 1 + 2 * f)
            cos_rows.append(d0 + 2 + 2 * f)
    return np.asarray(perm + sin_rows + cos_rows, np.int32)


def _dir_perm():
    r0, s0 = 0, 27
    perm = [r0, r0 + 1, r0 + 2, s0, s0 + 1, s0 + 2]
    sin_rows, cos_rows = [], []
    for f in range(4):
        sin_rows += [r0 + 3 + 6 * f + c for c in range(3)]
        sin_rows += [s0 + 3 + 6 * f + c for c in range(3)]
        cos_rows += [r0 + 6 + 6 * f + c for c in range(3)]
        cos_rows += [s0 + 6 + 6 * f + c for c in range(3)]
    return np.asarray(perm + sin_rows + cos_rows, np.int32)


_XYZ_PERM = _xyz_perm()
_DIR_PERM = _dir_perm()


def _shift_right(x, sh):
    z = jnp.zeros((x.shape[0], sh), x.dtype)
    return jnp.concatenate([z, x[:, :-sh]], axis=1)


def _fused(q_ref, rays_ref, pt_ref, tri_ref, mom_ref, mh_ref, ml_ref,
           ml2_ref, ro_ref,
           w0_ref, b0_ref, w1_ref, b1_ref, w2_ref, b2_ref, w3_ref, b3_ref,
           w4a_ref, w4b_ref, b4_ref, w5_ref, b5_ref, w6_ref, b6_ref,
           w7_ref, b7_ref, wf_ref, bf_ref, wdh_ref, wdd_ref, bd_ref,
           wrgb_ref, brgb_ref, out_ref):
    q = q_ref[...]
    qx = q[:, 0:1]
    qy = q[:, 1:2]
    qz = q[:, 2:3]

    tri = tri_ref[...]

    def chunk_body(c, carry):
        count, s_a, s_b, zcnt = carry
        done = jnp.min(count) >= KSEL
        return jax.lax.cond(done, lambda: carry, lambda: chunk_work(c, carry))

    def chunk_work(c, carry):
        count, s_a, s_b, zcnt = carry
        off = c * CK
        px = pt_ref[0:1, pl.ds(off, CK)]
        py = pt_ref[1:2, pl.ds(off, CK)]
        pz = pt_ref[2:3, pl.ds(off, CK)]
        dx = qx - px
        dy = qy - py
        dz = qz - pz
        d2 = (dx * dx + dy * dy) + dz * dz
        m = d2 < 1.0
        m_b = m.astype(jnp.bfloat16)
        cs = jnp.dot(m_b, tri, preferred_element_type=jnp.float32)
        keep = count + cs <= KSEL
        sel = (m & keep).astype(jnp.float32)
        w = sel * (1.0 - d2 * jnp.sqrt(d2))
        sel_b = sel.astype(jnp.bfloat16)
        mh = mh_ref[pl.ds(off, CK), :]
        ml = ml_ref[pl.ds(off, CK), :]
        ml2 = ml2_ref[pl.ds(off, CK), :]
        s_a = s_a + (jnp.dot(sel_b, mh, preferred_element_type=jnp.float32)
                     + jnp.dot(sel_b, ml, preferred_element_type=jnp.float32)
                     + jnp.dot(sel_b, ml2, preferred_element_type=jnp.float32))
        mom = mom_ref[pl.ds(off, CK), :]
        s_b = s_b + jnp.dot(w, mom, preferred_element_type=jnp.float32,
                            precision=_HI)
        zsel_b = (sel * (d2 == 0.0)).astype(jnp.bfloat16)
        zcnt = zcnt + jnp.dot(zsel_b, mh,
                              preferred_element_type=jnp.float32)[:, 6:7]
        count = count + cs[:, CK - 1:CK]
        return count, s_a, s_b, zcnt

    count, s_a, s_b, zcnt = jax.lax.fori_loop(
        0, NCK, chunk_body,
        (jnp.zeros((QB, 1), jnp.float32),
         jnp.zeros((QB, 8), jnp.float32),
         jnp.zeros((QB, 8), jnp.float32),
         jnp.zeros((QB, 1), jnp.float32)))

    n = s_a[:, 6:7]
    s_p = s_a[:, 0:3]
    s_p2 = s_a[:, 3:6]
    s_w = s_b[:, 6:7]
    s_wp3 = s_b[:, 0:3]

    den = s_w
    spos = s_wp3 / (s_w + 1e-12)
    s_d = s_p - n * q
    mean = s_d / (n + 1e-12)
    s_d2 = s_p2 - 2.0 * q * s_p + n * (q * q)
    var = (s_d2 - 2.0 * mean * s_d + n * (mean * mean)) / (n + 1e-12)
    ro = ro_ref[...]
    dirs = spos - ro
    dnorm = jnp.sqrt(jnp.sum(dirs * dirs, axis=1, keepdims=True))
    sdir = dirs / dnorm

    p10 = jnp.concatenate([q, spos, var, den], axis=1)
    p9 = p10[:, 0:9]
    scaled = jnp.concatenate(
        [p10 * (2.0 ** f) for f in range(4)]
        + [p9 * (2.0 ** f) for f in range(4, 10)], axis=1)
    fx = jnp.concatenate([p10, jnp.sin(scaled), jnp.cos(scaled)], axis=1)

    rq = rays_ref[...]
    d6 = jnp.concatenate([rq, sdir], axis=1)
    scaled_d = jnp.concatenate([d6 * (2.0 ** f) for f in range(4)], axis=1)
    fd = jnp.concatenate([d6, jnp.sin(scaled_d), jnp.cos(scaled_d)], axis=1)

    def mm(x, w):
        xh = x.astype(jnp.bfloat16)
        xl = (x - xh.astype(jnp.float32)).astype(jnp.bfloat16)
        wh = w.astype(jnp.bfloat16)
        wl = (w - wh.astype(jnp.float32)).astype(jnp.bfloat16)
        return (jnp.dot(xh, wh, preferred_element_type=jnp.float32)
                + jnp.dot(xl, wh, preferred_element_type=jnp.float32)
                + jnp.dot(xh, wl, preferred_element_type=jnp.float32))

    def lin(x, w_ref, b_ref):
        return mm(x, w_ref[...]) + b_ref[...]

    h = jnp.maximum(lin(fx, w0_ref, b0_ref), 0.0)
    h = jnp.maximum(lin(h, w1_ref, b1_ref), 0.0)
    h = jnp.maximum(lin(h, w2_ref, b2_ref), 0.0)
    h = jnp.maximum(lin(h, w3_ref, b3_ref), 0.0)
    h = jnp.maximum(lin(fx, w4a_ref, b4_ref) + mm(h, w4b_ref[...]), 0.0)
    h = jnp.maximum(lin(h, w5_ref, b5_ref), 0.0)
    h = jnp.maximum(lin(h, w6_ref, b6_ref), 0.0)
    h = jnp.maximum(lin(h, w7_ref, b7_ref), 0.0)
    hf = lin(h, wf_ref, bf_ref)
    dfeed = jnp.maximum(lin(hf, wdh_ref, bd_ref) + mm(fd, wdd_ref[...]), 0.0)
    rgb = jax.nn.sigmoid(lin(dfeed, wrgb_ref, brgb_ref))

    mask = ((count >= KSEL) & (zcnt == 0.0)).astype(jnp.float32)
    out_ref[...] = rgb * mask


def kernel(ray_particles_0, physical_particles, ro, rays, params):
    q0 = ray_particles_0.reshape(-1, 3)
    n_samples = ray_particles_0.shape[1]
    rays_q0 = jnp.repeat(rays, n_samples, axis=0)
    order = jnp.argsort(jnp.sum(q0 * q0, axis=1))
    inv_order = jnp.argsort(order)
    q = jnp.take(q0, order, axis=0)
    rays_q = jnp.take(rays_q0, order, axis=0)
    pt = physical_particles.T
    pp = physical_particles
    mom = jnp.concatenate(
        [pp, pp * pp, jnp.ones((NP, 1), jnp.float32),
         jnp.zeros((NP, 1), jnp.float32)], axis=1)
    mom_hi = mom.astype(jnp.bfloat16)
    mom_lo = (mom - mom_hi.astype(jnp.float32)).astype(jnp.bfloat16)
    mom_lo2 = (mom - mom_hi.astype(jnp.float32)
               - mom_lo.astype(jnp.float32)).astype(jnp.bfloat16)
    tri = jnp.asarray(np.triu(np.ones((CK, CK), np.float32)), jnp.bfloat16)
    ro2 = ro.reshape(1, 3)

    w0 = params['xyz_0_w'][_XYZ_PERM]
    w4 = params['xyz_4_w']
    w4a = w4[:198][_XYZ_PERM]
    w4b = w4[198:]
    wd = params['dir_w']
    wdh = wd[:256]
    wdd = wd[256:][_DIR_PERM]

    def b2d(b):
        return b.reshape(1, -1)

    weights = [
        w0, b2d(params['xyz_0_b']),
        params['xyz_1_w'], b2d(params['xyz_1_b']),
        params['xyz_2_w'], b2d(params['xyz_2_b']),
        params['xyz_3_w'], b2d(params['xyz_3_b']),
        w4a, w4b, b2d(params['xyz_4_b']),
        params['xyz_5_w'], b2d(params['xyz_5_b']),
        params['xyz_6_w'], b2d(params['xyz_6_b']),
        params['xyz_7_w'], b2d(params['xyz_7_b']),
        params['final_w'], b2d(params['final_b']),
        wdh, wdd, b2d(params['dir_b']),
        params['rgb_w'], b2d(params['rgb_b']),
    ]

    def const_spec(a):
        return pl.BlockSpec(a.shape, lambda i: (0,) * a.ndim)

    grid = (NQ // QB,)
    out = pl.pallas_call(
        _fused,
        grid=grid,
        in_specs=[
            pl.BlockSpec((QB, 3), lambda i: (i, 0)),
            pl.BlockSpec((QB, 3), lambda i: (i, 0)),
            const_spec(pt),
            const_spec(tri),
            const_spec(mom),
            const_spec(mom_hi),
            const_spec(mom_lo),
            const_spec(mom_lo2),
            const_spec(ro2),
        ] + [const_spec(a) for a in weights],
        out_specs=pl.BlockSpec((QB, 3), lambda i: (i, 0)),
        out_shape=jax.ShapeDtypeStruct((NQ, 3), jnp.float32),
    )(q, rays_q, pt, tri, mom, mom_hi, mom_lo, mom_lo2, ro2, *weights)
    return jnp.take(out, inv_order, axis=0)


# speedup vs baseline: 1.0775x; 1.0775x over previous
import numpy as np
import jax
import jax.numpy as jnp
from jax.experimental import pallas as pl

NQ = 16384
NP = 8192
QB = 1024
CK = 256
NCK = NP // CK
KSEL = 32.0

_HI = jax.lax.Precision.HIGHEST


def _xyz_perm():
    q0, d0, s0, v0 = 0, 63, 72, 135
    perm = [q0, q0 + 1, q0 + 2, s0, s0 + 1, s0 + 2, v0, v0 + 1, v0 + 2, d0]
    sin_rows, cos_rows = [], []
    for f in range(10):
        sin_rows += [q0 + 3 + 6 * f + c for c in range(3)]
        sin_rows += [s0 + 3 + 6 * f + c for c in range(3)]
        sin_rows += [v0 + 3 + 6 * f + c for c in range(3)]
        cos_rows += [q0 + 6 + 6 * f + c for c in range(3)]
        cos_rows += [s0 + 6 + 6 * f + c for c in range(3)]
        cos_rows += [v0 + 6 + 6 * f + c for c in range(3)]
        if f < 4:
            sin_rows.append(d0 + 1 + 2 * f)
            cos_rows.append(d0 + 2 + 2 * f)
    return np.asarray(perm + sin_rows + cos_rows, np.int32)


def _dir_perm():
    r0, s0 = 0, 27
    perm = [r0, r0 + 1, r0 + 2, s0, s0 + 1, s0 + 2]
    sin_rows, cos_rows = [], []
    for f in range(4):
        sin_rows += [r0 + 3 + 6 * f + c for c in range(3)]
        sin_rows += [s0 + 3 + 6 * f + c for c in range(3)]
        cos_rows += [r0 + 6 + 6 * f + c for c in range(3)]
        cos_rows += [s0 + 6 + 6 * f + c for c in range(3)]
    return np.asarray(perm + sin_rows + cos_rows, np.int32)


_XYZ_PERM = _xyz_perm()
_DIR_PERM = _dir_perm()


def _shift_right(x, sh):
    z = jnp.zeros((x.shape[0], sh), x.dtype)
    return jnp.concatenate([z, x[:, :-sh]], axis=1)


def _fused(q_ref, rays_ref, pt_ref, tri_ref, mh_ref, ml_ref,
           ml2_ref, ro_ref,
           w0_ref, b0_ref, w1_ref, b1_ref, w2_ref, b2_ref, w3_ref, b3_ref,
           w4a_ref, w4b_ref, b4_ref, w5_ref, b5_ref, w6_ref, b6_ref,
           w7_ref, b7_ref, wf_ref, bf_ref, wdh_ref, wdd_ref, bd_ref,
           wrgb_ref, brgb_ref, out_ref):
    q = q_ref[...]
    qx = q[:, 0:1]
    qy = q[:, 1:2]
    qz = q[:, 2:3]

    tri = tri_ref[...]

    def chunk_body(c, carry):
        count, s_a, s_b, zcnt = carry
        done = jnp.min(count) >= KSEL
        return jax.lax.cond(done, lambda: carry, lambda: chunk_work(c, carry))

    def chunk_work(c, carry):
        count, s_a, s_b, zcnt = carry
        off = c * CK
        px = pt_ref[0:1, pl.ds(off, CK)]
        py = pt_ref[1:2, pl.ds(off, CK)]
        pz = pt_ref[2:3, pl.ds(off, CK)]
        dx = qx - px
        dy = qy - py
        dz = qz - pz
        d2 = (dx * dx + dy * dy) + dz * dz
        m = d2 < 1.0
        m_b = m.astype(jnp.bfloat16)
        cs = jnp.dot(m_b, tri, preferred_element_type=jnp.float32)
        keep = count + cs <= KSEL
        sel = (m & keep).astype(jnp.float32)
        w = sel * (1.0 - d2 * jnp.sqrt(d2))
        sel_b = sel.astype(jnp.bfloat16)
        mh = mh_ref[pl.ds(off, CK), :]
        ml = ml_ref[pl.ds(off, CK), :]
        ml2 = ml2_ref[pl.ds(off, CK), :]
        s_a = s_a + (jnp.dot(sel_b, mh, preferred_element_type=jnp.float32)
                     + jnp.dot(sel_b, ml, preferred_element_type=jnp.float32)
                     + jnp.dot(sel_b, ml2, preferred_element_type=jnp.float32))
        w1 = w.astype(jnp.bfloat16)
        r1 = w - w1.astype(jnp.float32)
        w2 = r1.astype(jnp.bfloat16)
        w3 = (r1 - w2.astype(jnp.float32)).astype(jnp.bfloat16)
        s_b = s_b + (jnp.dot(w1, mh, preferred_element_type=jnp.float32)
                     + jnp.dot(w2, mh, preferred_element_type=jnp.float32)
                     + jnp.dot(w3, mh, preferred_element_type=jnp.float32)
                     + jnp.dot(w1, ml, preferred_element_type=jnp.float32)
                     + jnp.dot(w2, ml, preferred_element_type=jnp.float32)
                     + jnp.dot(w1, ml2, preferred_element_type=jnp.float32))
        zsel_b = (sel * (d2 == 0.0)).astype(jnp.bfloat16)
        zcnt = zcnt + jnp.dot(zsel_b, mh,
                              preferred_element_type=jnp.float32)[:, 6:7]
        count = count + cs[:, CK - 1:CK]
        return count, s_a, s_b, zcnt

    count, s_a, s_b, zcnt = jax.lax.fori_loop(
        0, NCK, chunk_body,
        (jnp.zeros((QB, 1), jnp.float32),
         jnp.zeros((QB, 8), jnp.float32),
         jnp.zeros((QB, 8), jnp.float32),
         jnp.zeros((QB, 1), jnp.float32)))

    n = s_a[:, 6:7]
    s_p = s_a[:, 0:3]
    s_p2 = s_a[:, 3:6]
    s_w = s_b[:, 6:7]
    s_wp3 = s_b[:, 0:3]

    den = s_w
    spos = s_wp3 / (s_w + 1e-12)
    s_d = s_p - n * q
    mean = s_d / (n + 1e-12)
    s_d2 = s_p2 - 2.0 * q * s_p + n * (q * q)
    var = (s_d2 - 2.0 * mean * s_d + n * (mean * mean)) / (n + 1e-12)
    ro = ro_ref[...]
    dirs = spos - ro
    dnorm = jnp.sqrt(jnp.sum(dirs * dirs, axis=1, keepdims=True))
    sdir = dirs / dnorm

    p10 = jnp.concatenate([q, spos, var, den], axis=1)
    p9 = p10[:, 0:9]
    scaled = jnp.concatenate(
        [p10 * (2.0 ** f) for f in range(4)]
        + [p9 * (2.0 ** f) for f in range(4, 10)], axis=1)
    fx = jnp.concatenate([p10, jnp.sin(scaled), jnp.cos(scaled)], axis=1)

    rq = rays_ref[...]
    d6 = jnp.concatenate([rq, sdir], axis=1)
    scaled_d = jnp.concatenate([d6 * (2.0 ** f) for f in range(4)], axis=1)
    fd = jnp.concatenate([d6, jnp.sin(scaled_d), jnp.cos(scaled_d)], axis=1)

    def mm(x, w):
        xh = x.astype(jnp.bfloat16)
        xl = (x - xh.astype(jnp.float32)).astype(jnp.bfloat16)
        wh = w.astype(jnp.bfloat16)
        wl = (w - wh.astype(jnp.float32)).astype(jnp.bfloat16)
        return (jnp.dot(xh, wh, preferred_element_type=jnp.float32)
                + jnp.dot(xl, wh, preferred_element_type=jnp.float32)
                + jnp.dot(xh, wl, preferred_element_type=jnp.float32))

    def lin(x, w_ref, b_ref):
        return mm(x, w_ref[...]) + b_ref[...]

    h = jnp.maximum(lin(fx, w0_ref, b0_ref), 0.0)
    h = jnp.maximum(lin(h, w1_ref, b1_ref), 0.0)
    h = jnp.maximum(lin(h, w2_ref, b2_ref), 0.0)
    h = jnp.maximum(lin(h, w3_ref, b3_ref), 0.0)
    h = jnp.maximum(lin(fx, w4a_ref, b4_ref) + mm(h, w4b_ref[...]), 0.0)
    h = jnp.maximum(lin(h, w5_ref, b5_ref), 0.0)
    h = jnp.maximum(lin(h, w6_ref, b6_ref), 0.0)
    h = jnp.maximum(lin(h, w7_ref, b7_ref), 0.0)
    hf = lin(h, wf_ref, bf_ref)
    dfeed = jnp.maximum(lin(hf, wdh_ref, bd_ref) + mm(fd, wdd_ref[...]), 0.0)
    rgb = jax.nn.sigmoid(lin(dfeed, wrgb_ref, brgb_ref))

    mask = ((count >= KSEL) & (zcnt == 0.0)).astype(jnp.float32)
    out_ref[...] = rgb * mask


def kernel(ray_particles_0, physical_particles, ro, rays, params):
    q0 = ray_particles_0.reshape(-1, 3)
    n_samples = ray_particles_0.shape[1]
    rays_q0 = jnp.repeat(rays, n_samples, axis=0)
    order = jnp.argsort(jnp.sum(q0 * q0, axis=1))
    inv_order = jnp.argsort(order)
    q = jnp.take(q0, order, axis=0)
    rays_q = jnp.take(rays_q0, order, axis=0)
    pt = physical_particles.T
    pp = physical_particles
    mom = jnp.concatenate(
        [pp, pp * pp, jnp.ones((NP, 1), jnp.float32),
         jnp.zeros((NP, 1), jnp.float32)], axis=1)
    mom_hi = mom.astype(jnp.bfloat16)
    mom_lo = (mom - mom_hi.astype(jnp.float32)).astype(jnp.bfloat16)
    mom_lo2 = (mom - mom_hi.astype(jnp.float32)
               - mom_lo.astype(jnp.float32)).astype(jnp.bfloat16)
    tri = jnp.asarray(np.triu(np.ones((CK, CK), np.float32)), jnp.bfloat16)
    ro2 = ro.reshape(1, 3)

    w0 = params['xyz_0_w'][_XYZ_PERM]
    w4 = params['xyz_4_w']
    w4a = w4[:198][_XYZ_PERM]
    w4b = w4[198:]
    wd = params['dir_w']
    wdh = wd[:256]
    wdd = wd[256:][_DIR_PERM]

    def b2d(b):
        return b.reshape(1, -1)

    weights = [
        w0, b2d(params['xyz_0_b']),
        params['xyz_1_w'], b2d(params['xyz_1_b']),
        params['xyz_2_w'], b2d(params['xyz_2_b']),
        params['xyz_3_w'], b2d(params['xyz_3_b']),
        w4a, w4b, b2d(params['xyz_4_b']),
        params['xyz_5_w'], b2d(params['xyz_5_b']),
        params['xyz_6_w'], b2d(params['xyz_6_b']),
        params['xyz_7_w'], b2d(params['xyz_7_b']),
        params['final_w'], b2d(params['final_b']),
        wdh, wdd, b2d(params['dir_b']),
        params['rgb_w'], b2d(params['rgb_b']),
    ]

    def const_spec(a):
        return pl.BlockSpec(a.shape, lambda i: (0,) * a.ndim)

    grid = (NQ // QB,)
    out = pl.pallas_call(
        _fused,
        grid=grid,
        in_specs=[
            pl.BlockSpec((QB, 3), lambda i: (i, 0)),
            pl.BlockSpec((QB, 3), lambda i: (i, 0)),
            const_spec(pt),
            const_spec(tri),
            const_spec(mom_hi),
            const_spec(mom_lo),
            const_spec(mom_lo2),
            const_spec(ro2),
        ] + [const_spec(a) for a in weights],
        out_specs=pl.BlockSpec((QB, 3), lambda i: (i, 0)),
        out_shape=jax.ShapeDtypeStruct((NQ, 3), jnp.float32),
    )(q, rays_q, pt, tri, mom_hi, mom_lo, mom_lo2, ro2, *weights)
    return jnp.take(out, inv_order, axis=0)
